# baseline (speedup 1.0000x reference)
import functools

import jax
import jax.numpy as jnp
from jax import lax
from jax.experimental import pallas as pl
from jax.experimental.pallas import tpu as pltpu
from jax.experimental.pallas import tpu_sc as plsc

B, N, E, D, DE, S, H = 4, 1024, 65536, 128, 16, 1024, 8
PH = D // H

NC, NS, L = 2, 16, 16
NW = NC * NS
CH = 128

_F32 = jnp.float32


def _gelu(x):
    return 0.5 * x * (1.0 + lax.erf(x * (2.0 ** -0.5)))


def _layernorm(x, g, b, eps=1e-3):
    m = jnp.mean(x, axis=-1, keepdims=True)
    xc = x - m
    v = jnp.mean(xc * xc, axis=-1, keepdims=True)
    return xc * lax.rsqrt(v + eps) * g + b


def _proj_body(nf_ref, w_ref, p1_ref, p2_ref):
    nf = nf_ref[...]
    w1 = w_ref[0:D, :]
    w2 = w_ref[D:2 * D, :]
    p1_ref[...] = jnp.dot(nf, w1, preferred_element_type=_F32)
    p2_ref[...] = jnp.dot(nf, w2, preferred_element_type=_F32)


def _proj(nf, W_msg):
    return pl.pallas_call(
        _proj_body,
        out_shape=(jax.ShapeDtypeStruct((B * N, D), _F32),
                   jax.ShapeDtypeStruct((B * N, D), _F32)),
        name="proj_nodes",
    )(nf, W_msg)


def _sc_gather_body(p1_hbm, p2_hbm, edges_hbm, gsum_hbm,
                    idxa1, idxa2, pairb, rows1, rows2,
                    gsem0, gsem1, osem0, osem1):
    c = lax.axis_index("c")
    s = lax.axis_index("s")
    wid = c * NS + s
    epw = E // NW
    cpb = epw // CH
    nchunks = B * cpb
    gsems = (gsem0, gsem1)
    osems = (osem0, osem1)

    ev = lax.iota(jnp.int32, L) * 2
    od = ev + 1
    vpb = epw // L
    for b in range(B):
        pltpu.sync_copy(edges_hbm.at[pl.ds(2 * (b * E + wid * epw), 2 * epw)],
                        pairb)

        @plsc.parallel_loop(0, vpb, unroll=4)
        def _deint(j):
            sl = pl.ds(b * epw + j * L, L)
            idxa1[sl] = plsc.load_gather(pairb, [ev + j * (2 * L)]) + b * N
            idxa2[sl] = plsc.load_gather(pairb, [od + j * (2 * L)]) + b * N

    def prep(i, k, wait_out):
        if wait_out:
            pltpu.make_async_copy(rows1.at[k], gsum_hbm.at[pl.ds(0, CH)],
                                  osems[k]).wait()
        pltpu.async_copy(p1_hbm.at[idxa1.at[pl.ds(i * CH, CH)]],
                         rows1.at[k], gsems[k])
        pltpu.async_copy(p2_hbm.at[idxa2.at[pl.ds(i * CH, CH)]],
                         rows2.at[k], gsems[k])

    def consume(i, k):
        pltpu.make_async_copy(p1_hbm.at[idxa1.at[pl.ds(0, CH)]],
                              rows1.at[k], gsems[k]).wait()
        pltpu.make_async_copy(p2_hbm.at[idxa2.at[pl.ds(0, CH)]],
                              rows2.at[k], gsems[k]).wait()

        @plsc.parallel_loop(0, CH, unroll=2)
        def _addrow(r):
            for j in range(D // L):
                sl = pl.ds(j * L, L)
                rows1[k, r, sl] = rows1[k, r, sl] + rows2[k, r, sl]

        b = i // cpb
        base = b * E + wid * epw + (i - b * cpb) * CH
        pltpu.async_copy(rows1.at[k], gsum_hbm.at[pl.ds(base, CH)], osems[k])

    prep(0, 0, False)
    prep(1, 1, False)

    def step(si, carry):
        c0 = 2 * si
        consume(c0, 0)

        @pl.when(si < nchunks // 2 - 1)
        def _():
            prep(c0 + 2, 0, True)

        consume(c0 + 1, 1)

        @pl.when(si < nchunks // 2 - 1)
        def _():
            prep(c0 + 3, 1, True)

        return carry

    lax.fori_loop(0, nchunks // 2, step, 0)
    pltpu.make_async_copy(rows1.at[0], gsum_hbm.at[pl.ds(0, CH)], osem0).wait()
    pltpu.make_async_copy(rows1.at[1], gsum_hbm.at[pl.ds(0, CH)], osem1).wait()


def _sc_gather(p1, p2, edges_flat):
    mesh = plsc.VectorSubcoreMesh(core_axis_name="c", subcore_axis_name="s")
    return pl.kernel(
        _sc_gather_body,
        out_type=jax.ShapeDtypeStruct((B * E, D), _F32),
        mesh=mesh,
        scratch_types=[
            pltpu.VMEM((B * (E // NW),), jnp.int32),
            pltpu.VMEM((B * (E // NW),), jnp.int32),
            pltpu.VMEM((2 * (E // NW),), jnp.int32),
            pltpu.VMEM((2, CH, D), _F32),
            pltpu.VMEM((2, CH, D), _F32),
            pltpu.SemaphoreType.DMA,
            pltpu.SemaphoreType.DMA,
            pltpu.SemaphoreType.DMA,
            pltpu.SemaphoreType.DMA,
        ],
        compiler_params=pltpu.CompilerParams(needs_layout_passes=False),
        name="sc_gather_sum",
    )(p1, p2, edges_flat)


_BE = 4096


def _edge_mlp_body(g_ref, ef_ref, w_ref, bm_ref, g1_ref, b1_ref, ew_ref, out_ref):
    w3 = w_ref[2 * D:2 * D + DE, :]
    x = g_ref[...] + jnp.dot(ef_ref[...], w3, preferred_element_type=_F32)
    x = x + bm_ref[...]
    x = _gelu(x)
    x = _layernorm(x, g1_ref[...], b1_ref[...])
    li2 = lax.broadcasted_iota(jnp.int32, (_BE, 2), 1)
    wcol = jnp.sum(jnp.where(li2 == 1, ew_ref[...], 0.0),
                   axis=-1, keepdims=True)
    out_ref[...] = x * wcol


def _edge_mlp(gsum, eff, W_msg, b_msg, ln1_g, ln1_b, ew2):
    grid = (B * E // _BE,)
    return pl.pallas_call(
        _edge_mlp_body,
        grid=grid,
        in_specs=[
            pl.BlockSpec((_BE, D), lambda i: (i, 0)),
            pl.BlockSpec((_BE, DE), lambda i: (i, 0)),
            pl.BlockSpec((2 * D + DE, D), lambda i: (0, 0)),
            pl.BlockSpec((D,), lambda i: (0,)),
            pl.BlockSpec((D,), lambda i: (0,)),
            pl.BlockSpec((D,), lambda i: (0,)),
            pl.BlockSpec((_BE, 2), lambda i: (i, 0)),
        ],
        out_specs=pl.BlockSpec((_BE, D), lambda i: (i, 0)),
        out_shape=jax.ShapeDtypeStruct((B * E, D), _F32),
        name="edge_mlp",
    )(gsum, eff, W_msg, b_msg, ln1_g, ln1_b, ew2)


def _sc_scatter_body(w_hbm, edges_hbm, ew_hbm, agg_hbm,
                     acc, dstt, dpair, ewb0, ewb1, comb, rows, zb,
                     rsem0, rsem1, ssem0, ssem1):
    ewbs = (ewb0, ewb1)
    c = lax.axis_index("c")
    sid = lax.axis_index("s")
    rsems = (rsem0, rsem1)
    ssems = (ssem0, ssem1)
    ev = lax.iota(jnp.int32, L) * 2
    od = ev + 1

    def zrow(r, carry):
        for k in range(D // L):
            zb[r, pl.ds(k * L, L)] = jnp.zeros((L,), _F32)
        return carry

    lax.fori_loop(0, 64, zrow, 0)
    pltpu.sync_copy(zb, acc.at[pl.ds(sid * 128, 64)])
    pltpu.sync_copy(zb, acc.at[pl.ds(sid * 128 + 64, 64)])
    plsc.subcore_barrier()

    eps = E // NS
    cpb = eps // CH
    for t in range(2):
        b = 2 * c + t
        pltpu.sync_copy(edges_hbm.at[pl.ds(2 * b * E, 2 * S)], dpair)

        @plsc.parallel_loop(0, S // L, unroll=4)
        def _dstt(j):
            dstt[pl.ds(j * L, L)] = plsc.load_gather(dpair, [od + j * (2 * L)])

        def base_of(ci):
            return b * E + sid * eps + ci * CH

        def prep(ci, k, wait_scatter):
            if wait_scatter:
                pltpu.make_async_copy(rows.at[k], acc.at[comb.at[k]],
                                      ssems[k]).wait()
            base = base_of(ci)
            pltpu.sync_copy(ew_hbm.at[pl.ds(2 * base, 2 * CH)], ewbs[k])
            pltpu.async_copy(w_hbm.at[pl.ds(base, CH)], rows.at[k], rsems[k])

        def consume(k):
            for j in range(CH // L):
                sl = pl.ds(j * L, L)
                sv = plsc.load_gather(ewbs[k], [ev + j * (2 * L)])
                cv = plsc.load_gather(dstt, [sv.astype(jnp.int32)])
                comb[k, sl] = cv + (t * N)
            pltpu.make_async_copy(w_hbm.at[pl.ds(0, CH)], rows.at[k],
                                  rsems[k]).wait()
            pltpu.async_copy(rows.at[k], acc.at[comb.at[k]], ssems[k], add=True)

        prep(0, 0, False)
        prep(1, 1, False)

        def step(si, carry):
            consume(0)

            @pl.when(si < cpb // 2 - 1)
            def _():
                prep(2 * si + 2, 0, True)

            consume(1)

            @pl.when(si < cpb // 2 - 1)
            def _():
                prep(2 * si + 3, 1, True)

            return carry

        lax.fori_loop(0, cpb // 2, step, 0)
        pltpu.make_async_copy(rows.at[0], acc.at[comb.at[0]], ssem0).wait()
        pltpu.make_async_copy(rows.at[1], acc.at[comb.at[1]], ssem1).wait()

    plsc.subcore_barrier()
    pltpu.sync_copy(acc.at[pl.ds(sid * 128, 128)], rows.at[0])
    pltpu.sync_copy(rows.at[0], agg_hbm.at[pl.ds(2 * c * N + sid * 128, 128)])


def _sc_scatter(wflat, edges_flat, ew_flat):
    mesh = plsc.VectorSubcoreMesh(core_axis_name="c", subcore_axis_name="s")
    return pl.kernel(
        _sc_scatter_body,
        out_type=jax.ShapeDtypeStruct((B * N, D), _F32),
        mesh=mesh,
        scratch_types=[
            pltpu.VMEM_SHARED((2 * N, D), _F32),
            pltpu.VMEM((S,), jnp.int32),
            pltpu.VMEM((2 * S,), jnp.int32),
            pltpu.VMEM((2 * CH,), _F32),
            pltpu.VMEM((2 * CH,), _F32),
            pltpu.VMEM((2, CH), jnp.int32),
            pltpu.VMEM((2, CH, D), _F32),
            pltpu.VMEM((64, D), _F32),
            pltpu.SemaphoreType.DMA,
            pltpu.SemaphoreType.DMA,
            pltpu.SemaphoreType.DMA,
            pltpu.SemaphoreType.DMA,
        ],
        compiler_params=pltpu.CompilerParams(needs_layout_passes=False),
        name="sc_scatter_agg",
    )(wflat, edges_flat, ew_flat)


def _attn_body(n_ref, a_ref, wq_ref, bq_ref, wk_ref, bk_ref, wv_ref, bv_ref,
               wc_ref, bc_ref, g2_ref, b2_ref, out_ref):
    xc = jnp.concatenate([n_ref[0], a_ref[0]], axis=1)
    q = jnp.dot(xc, wq_ref[...], preferred_element_type=_F32) + bq_ref[...]
    k = jnp.dot(xc, wk_ref[...], preferred_element_type=_F32) + bk_ref[...]
    v = jnp.dot(xc, wv_ref[...], preferred_element_type=_F32) + bv_ref[...]
    li = lax.broadcasted_iota(jnp.int32, (N, D), 1)
    att = jnp.zeros((N, D), _F32)
    inv_sqrt = 1.0 / (PH ** 0.5)
    for h in range(H):
        mh = (li // PH) == h
        qm = jnp.where(mh, q, 0.0)
        score = lax.dot_general(qm, k, (((1,), (1,)), ((), ())),
                                preferred_element_type=_F32) * inv_sqrt
        mx = jnp.max(score, axis=-1, keepdims=True)
        p = jnp.exp(score - mx)
        attw = p / jnp.sum(p, axis=-1, keepdims=True)
        vm = jnp.where(mh, v, 0.0)
        att = att + lax.dot_general(attw, vm, (((1,), (0,)), ((), ())),
                                    preferred_element_type=_F32)
    u = jnp.dot(att, wc_ref[...], preferred_element_type=_F32) + bc_ref[...]
    u = _gelu(u)
    out_ref[0] = _layernorm(u, g2_ref[...], b2_ref[...])


def _attention(nodes, agg, Wq, bq, Wk, bk, Wv, bv, Wc, bc, ln2_g, ln2_b):
    vec = lambda: pl.BlockSpec((D,), lambda i: (0,))
    mat2 = lambda: pl.BlockSpec((2 * D, D), lambda i: (0, 0))
    return pl.pallas_call(
        _attn_body,
        grid=(B,),
        in_specs=[
            pl.BlockSpec((1, N, D), lambda i: (i, 0, 0)),
            pl.BlockSpec((1, N, D), lambda i: (i, 0, 0)),
            mat2(), vec(), mat2(), vec(), mat2(), vec(),
            pl.BlockSpec((D, D), lambda i: (0, 0)), vec(),
            vec(), vec(),
        ],
        out_specs=pl.BlockSpec((1, N, D), lambda i: (i, 0, 0)),
        out_shape=jax.ShapeDtypeStruct((B, N, D), _F32),
        name="attention_update",
    )(nodes, agg, Wq, bq, Wk, bk, Wv, bv, Wc, bc, ln2_g, ln2_b)


def kernel(nodes, edge_features, edges, edge_weights, W_msg, b_msg,
           ln1_g, ln1_b, Wq, bq, Wk, bk, Wv, bv, Wc, bc, ln2_g, ln2_b):
    edges_flat = edges.astype(jnp.int32).reshape(B * E * 2)
    ew_flat = edge_weights.reshape(B * E * 2)
    ew2 = edge_weights.reshape(B * E, 2)
    eff = edge_features.reshape(B * E, DE)
    nf = nodes.reshape(B * N, D)

    p1, p2 = _proj(nf, W_msg)
    gsum = _sc_gather(p1, p2, edges_flat)
    wflat = _edge_mlp(gsum, eff, W_msg, b_msg, ln1_g, ln1_b, ew2)
    agg = _sc_scatter(wflat, edges_flat, ew_flat)
    updated = _attention(nodes, agg.reshape(B, N, D),
                         Wq, bq, Wk, bk, Wv, bv, Wc, bc, ln2_g, ln2_b)
    return (updated, wflat.reshape(B, E, D), edges, edge_weights)


# speedup vs baseline: 1.2844x; 1.2844x over previous
import functools

import jax
import jax.numpy as jnp
from jax import lax
from jax.experimental import pallas as pl
from jax.experimental.pallas import tpu as pltpu
from jax.experimental.pallas import tpu_sc as plsc

B, N, E, D, DE, S, H = 4, 1024, 65536, 128, 16, 1024, 8
PH = D // H

NC, NS, L = 2, 16, 16
NW = NC * NS
CH = 128

_F32 = jnp.float32


def _gelu(x):
    return 0.5 * x * (1.0 + lax.erf(x * (2.0 ** -0.5)))


def _layernorm(x, g, b, eps=1e-3):
    m = jnp.mean(x, axis=-1, keepdims=True)
    xc = x - m
    v = jnp.mean(xc * xc, axis=-1, keepdims=True)
    return xc * lax.rsqrt(v + eps) * g + b


def _proj_body(nf_ref, w_ref, p1_ref, p2_ref):
    nf = nf_ref[...]
    w1 = w_ref[0:D, :]
    w2 = w_ref[D:2 * D, :]
    p1_ref[...] = jnp.dot(nf, w1, preferred_element_type=_F32)
    p2_ref[...] = jnp.dot(nf, w2, preferred_element_type=_F32)


def _proj(nf, W_msg):
    return pl.pallas_call(
        _proj_body,
        out_shape=(jax.ShapeDtypeStruct((B * N, D), _F32),
                   jax.ShapeDtypeStruct((B * N, D), _F32)),
        name="proj_nodes",
    )(nf, W_msg)


def _sc_gather_body(p1_hbm, p2_hbm, src_hbm, dst_hbm, gsum_hbm,
                    idxa1, idxa2, rows1, rows2, gsem0, gsem1, osem0, osem1):
    c = lax.axis_index("c")
    s = lax.axis_index("s")
    wid = c * NS + s
    epw = E // NW
    cpb = epw // CH
    nchunks = B * cpb
    gsems = (gsem0, gsem1)
    osems = (osem0, osem1)

    for b in range(B):
        pltpu.sync_copy(src_hbm.at[pl.ds(b * E + wid * epw, epw)],
                        idxa1.at[pl.ds(b * epw, epw)])
        pltpu.sync_copy(dst_hbm.at[pl.ds(b * E + wid * epw, epw)],
                        idxa2.at[pl.ds(b * epw, epw)])
    vpb = epw // L

    @plsc.parallel_loop(vpb, B * vpb, unroll=4)
    def _offset(j):
        boff = (j // vpb) * N
        sl = pl.ds(j * L, L)
        idxa1[sl] = idxa1[sl] + boff
        idxa2[sl] = idxa2[sl] + boff

    def prep(i, k, wait_out):
        if wait_out:
            pltpu.make_async_copy(rows1.at[k], gsum_hbm.at[pl.ds(0, CH)],
                                  osems[k]).wait()
        pltpu.async_copy(p1_hbm.at[idxa1.at[pl.ds(i * CH, CH)]],
                         rows1.at[k], gsems[k])
        pltpu.async_copy(p2_hbm.at[idxa2.at[pl.ds(i * CH, CH)]],
                         rows2.at[k], gsems[k])

    def consume(i, k):
        pltpu.make_async_copy(p1_hbm.at[idxa1.at[pl.ds(0, CH)]],
                              rows1.at[k], gsems[k]).wait()
        pltpu.make_async_copy(p2_hbm.at[idxa2.at[pl.ds(0, CH)]],
                              rows2.at[k], gsems[k]).wait()

        @plsc.parallel_loop(0, CH, unroll=2)
        def _addrow(r):
            for j in range(D // L):
                sl = pl.ds(j * L, L)
                rows1[k, r, sl] = rows1[k, r, sl] + rows2[k, r, sl]

        b = i // cpb
        base = b * E + wid * epw + (i - b * cpb) * CH
        pltpu.async_copy(rows1.at[k], gsum_hbm.at[pl.ds(base, CH)], osems[k])

    prep(0, 0, False)
    prep(1, 1, False)

    def step(si, carry):
        c0 = 2 * si
        consume(c0, 0)

        @pl.when(si < nchunks // 2 - 1)
        def _():
            prep(c0 + 2, 0, True)

        consume(c0 + 1, 1)

        @pl.when(si < nchunks // 2 - 1)
        def _():
            prep(c0 + 3, 1, True)

        return carry

    lax.fori_loop(0, nchunks // 2, step, 0)
    pltpu.make_async_copy(rows1.at[0], gsum_hbm.at[pl.ds(0, CH)], osem0).wait()
    pltpu.make_async_copy(rows1.at[1], gsum_hbm.at[pl.ds(0, CH)], osem1).wait()


def _sc_gather(p1, p2, src, dst):
    mesh = plsc.VectorSubcoreMesh(core_axis_name="c", subcore_axis_name="s")
    return pl.kernel(
        _sc_gather_body,
        out_type=jax.ShapeDtypeStruct((B * E, D), _F32),
        mesh=mesh,
        scratch_types=[
            pltpu.VMEM((B * (E // NW),), jnp.int32),
            pltpu.VMEM((B * (E // NW),), jnp.int32),
            pltpu.VMEM((2, CH, D), _F32),
            pltpu.VMEM((2, CH, D), _F32),
            pltpu.SemaphoreType.DMA,
            pltpu.SemaphoreType.DMA,
            pltpu.SemaphoreType.DMA,
            pltpu.SemaphoreType.DMA,
        ],
        compiler_params=pltpu.CompilerParams(needs_layout_passes=False),
        name="sc_gather_sum",
    )(p1, p2, src, dst)


_BE = 4096


def _edge_mlp_body(g_ref, ef_ref, w_ref, bm_ref, g1_ref, b1_ref, wc_ref, out_ref):
    w3 = w_ref[2 * D:2 * D + DE, :]
    x = g_ref[...] + jnp.dot(ef_ref[...], w3, preferred_element_type=_F32)
    x = x + bm_ref[...]
    x = _gelu(x)
    x = _layernorm(x, g1_ref[...], b1_ref[...])
    out_ref[...] = x * wc_ref[...]


def _edge_mlp(gsum, eff, W_msg, b_msg, ln1_g, ln1_b, wcol):
    grid = (B * E // _BE,)
    return pl.pallas_call(
        _edge_mlp_body,
        grid=grid,
        in_specs=[
            pl.BlockSpec((_BE, D), lambda i: (i, 0)),
            pl.BlockSpec((_BE, DE), lambda i: (i, 0)),
            pl.BlockSpec((2 * D + DE, D), lambda i: (0, 0)),
            pl.BlockSpec((D,), lambda i: (0,)),
            pl.BlockSpec((D,), lambda i: (0,)),
            pl.BlockSpec((D,), lambda i: (0,)),
            pl.BlockSpec((_BE, 1), lambda i: (i, 0)),
        ],
        out_specs=pl.BlockSpec((_BE, D), lambda i: (i, 0)),
        out_shape=jax.ShapeDtypeStruct((B * E, D), _F32),
        name="edge_mlp",
    )(gsum, eff, W_msg, b_msg, ln1_g, ln1_b, wcol)


def _sc_scatter_body(w_hbm, seg_hbm, dst2_hbm, agg_hbm,
                     acc, dstt, segb, comb, rows, zb,
                     rsem0, rsem1, ssem0, ssem1):
    c = lax.axis_index("c")
    sid = lax.axis_index("s")
    rsems = (rsem0, rsem1)
    ssems = (ssem0, ssem1)

    def zrow(r, carry):
        for k in range(D // L):
            zb[r, pl.ds(k * L, L)] = jnp.zeros((L,), _F32)
        return carry

    lax.fori_loop(0, 64, zrow, 0)
    pltpu.sync_copy(zb, acc.at[pl.ds(sid * 128, 64)])
    pltpu.sync_copy(zb, acc.at[pl.ds(sid * 128 + 64, 64)])
    plsc.subcore_barrier()

    eps = E // NS
    cpb = eps // CH
    for t in range(2):
        b = 2 * c + t
        pltpu.sync_copy(dst2_hbm.at[pl.ds(b * S, S)], dstt)

        def base_of(ci):
            return b * E + sid * eps + ci * CH

        def prep(ci, k, wait_scatter):
            if wait_scatter:
                pltpu.make_async_copy(rows.at[k], acc.at[comb.at[k]],
                                      ssems[k]).wait()
            base = base_of(ci)
            pltpu.sync_copy(seg_hbm.at[pl.ds(base, CH)], segb.at[k])
            pltpu.async_copy(w_hbm.at[pl.ds(base, CH)], rows.at[k], rsems[k])

        def consume(k):
            for j in range(CH // L):
                sl = pl.ds(j * L, L)
                sv = segb[k, sl]
                cv = plsc.load_gather(dstt, [sv])
                comb[k, sl] = cv + (t * N)
            pltpu.make_async_copy(w_hbm.at[pl.ds(0, CH)], rows.at[k],
                                  rsems[k]).wait()
            pltpu.async_copy(rows.at[k], acc.at[comb.at[k]], ssems[k], add=True)

        prep(0, 0, False)
        prep(1, 1, False)

        def step(si, carry):
            consume(0)

            @pl.when(si < cpb // 2 - 1)
            def _():
                prep(2 * si + 2, 0, True)

            consume(1)

            @pl.when(si < cpb // 2 - 1)
            def _():
                prep(2 * si + 3, 1, True)

            return carry

        lax.fori_loop(0, cpb // 2, step, 0)
        pltpu.make_async_copy(rows.at[0], acc.at[comb.at[0]], ssem0).wait()
        pltpu.make_async_copy(rows.at[1], acc.at[comb.at[1]], ssem1).wait()

    plsc.subcore_barrier()
    pltpu.sync_copy(acc.at[pl.ds(sid * 128, 128)], rows.at[0])
    pltpu.sync_copy(rows.at[0], agg_hbm.at[pl.ds(2 * c * N + sid * 128, 128)])


def _sc_scatter(wflat, seg, dst2):
    mesh = plsc.VectorSubcoreMesh(core_axis_name="c", subcore_axis_name="s")
    return pl.kernel(
        _sc_scatter_body,
        out_type=jax.ShapeDtypeStruct((B * N, D), _F32),
        mesh=mesh,
        scratch_types=[
            pltpu.VMEM_SHARED((2 * N, D), _F32),
            pltpu.VMEM((S,), jnp.int32),
            pltpu.VMEM((2, CH), jnp.int32),
            pltpu.VMEM((2, CH), jnp.int32),
            pltpu.VMEM((2, CH, D), _F32),
            pltpu.VMEM((64, D), _F32),
            pltpu.SemaphoreType.DMA,
            pltpu.SemaphoreType.DMA,
            pltpu.SemaphoreType.DMA,
            pltpu.SemaphoreType.DMA,
        ],
        compiler_params=pltpu.CompilerParams(needs_layout_passes=False),
        name="sc_scatter_agg",
    )(wflat, seg, dst2)


def _attn_body(n_ref, a_ref, wq_ref, bq_ref, wk_ref, bk_ref, wv_ref, bv_ref,
               wc_ref, bc_ref, g2_ref, b2_ref, out_ref):
    xc = jnp.concatenate([n_ref[0], a_ref[0]], axis=1)
    q = jnp.dot(xc, wq_ref[...], preferred_element_type=_F32) + bq_ref[...]
    k = jnp.dot(xc, wk_ref[...], preferred_element_type=_F32) + bk_ref[...]
    v = jnp.dot(xc, wv_ref[...], preferred_element_type=_F32) + bv_ref[...]
    li = lax.broadcasted_iota(jnp.int32, (N, D), 1)
    att = jnp.zeros((N, D), _F32)
    inv_sqrt = 1.0 / (PH ** 0.5)
    for h in range(H):
        mh = (li // PH) == h
        qm = jnp.where(mh, q, 0.0)
        score = lax.dot_general(qm, k, (((1,), (1,)), ((), ())),
                                preferred_element_type=_F32) * inv_sqrt
        mx = jnp.max(score, axis=-1, keepdims=True)
        p = jnp.exp(score - mx)
        attw = p / jnp.sum(p, axis=-1, keepdims=True)
        vm = jnp.where(mh, v, 0.0)
        att = att + lax.dot_general(attw, vm, (((1,), (0,)), ((), ())),
                                    preferred_element_type=_F32)
    u = jnp.dot(att, wc_ref[...], preferred_element_type=_F32) + bc_ref[...]
    u = _gelu(u)
    out_ref[0] = _layernorm(u, g2_ref[...], b2_ref[...])


def _attention(nodes, agg, Wq, bq, Wk, bk, Wv, bv, Wc, bc, ln2_g, ln2_b):
    vec = lambda: pl.BlockSpec((D,), lambda i: (0,))
    mat2 = lambda: pl.BlockSpec((2 * D, D), lambda i: (0, 0))
    return pl.pallas_call(
        _attn_body,
        grid=(B,),
        in_specs=[
            pl.BlockSpec((1, N, D), lambda i: (i, 0, 0)),
            pl.BlockSpec((1, N, D), lambda i: (i, 0, 0)),
            mat2(), vec(), mat2(), vec(), mat2(), vec(),
            pl.BlockSpec((D, D), lambda i: (0, 0)), vec(),
            vec(), vec(),
        ],
        out_specs=pl.BlockSpec((1, N, D), lambda i: (i, 0, 0)),
        out_shape=jax.ShapeDtypeStruct((B, N, D), _F32),
        name="attention_update",
    )(nodes, agg, Wq, bq, Wk, bk, Wv, bv, Wc, bc, ln2_g, ln2_b)


def kernel(nodes, edge_features, edges, edge_weights, W_msg, b_msg,
           ln1_g, ln1_b, Wq, bq, Wk, bk, Wv, bv, Wc, bc, ln2_g, ln2_b):
    e32 = edges.astype(jnp.int32)
    src = e32[:, :, 0].reshape(B * E)
    dst = e32[:, :, 1].reshape(B * E)
    dst2 = e32[:, :S, 1].reshape(B * S)
    seg = edge_weights[:, :, 0].astype(jnp.int32).reshape(B * E)
    wcol = edge_weights[:, :, 1].reshape(B * E, 1)
    eff = edge_features.reshape(B * E, DE)
    nf = nodes.reshape(B * N, D)

    p1, p2 = _proj(nf, W_msg)
    gsum = _sc_gather(p1, p2, src, dst)
    wflat = _edge_mlp(gsum, eff, W_msg, b_msg, ln1_g, ln1_b, wcol)
    agg = _sc_scatter(wflat, seg, dst2)
    updated = _attention(nodes, agg.reshape(B, N, D),
                         Wq, bq, Wk, bk, Wv, bv, Wc, bc, ln2_g, ln2_b)
    return (updated, wflat.reshape(B, E, D), edges, edge_weights)


# speedup vs baseline: 1.2891x; 1.0037x over previous
import jax
import jax.numpy as jnp
from jax import lax
from jax.experimental import pallas as pl
from jax.experimental.pallas import tpu as pltpu
from jax.experimental.pallas import tpu_sc as plsc

B, N, E, D, DE, S, H = 4, 1024, 65536, 128, 16, 1024, 8
PH = D // H

NC, NS, L = 2, 16, 16
NW = NC * NS
CH = 128

_F32 = jnp.float32


def _gelu(x):
    return 0.5 * x * (1.0 + lax.erf(x * (2.0 ** -0.5)))


def _layernorm(x, g, b, eps=1e-3):
    m = jnp.mean(x, axis=-1, keepdims=True)
    xc = x - m
    v = jnp.mean(xc * xc, axis=-1, keepdims=True)
    return xc * lax.rsqrt(v + eps) * g + b


def _proj_body(nf_ref, w_ref, p1_ref, p2_ref):
    nf = nf_ref[...]
    w1 = w_ref[0:D, :]
    w2 = w_ref[D:2 * D, :]
    p1_ref[...] = jnp.dot(nf, w1, preferred_element_type=_F32)
    p2_ref[...] = jnp.dot(nf, w2, preferred_element_type=_F32)


def _proj(nf, W_msg):
    return pl.pallas_call(
        _proj_body,
        out_shape=(jax.ShapeDtypeStruct((B * N, D), _F32),
                   jax.ShapeDtypeStruct((B * N, D), _F32)),
        name="proj_nodes",
    )(nf, W_msg)


def _sc_gather_body(p1_hbm, p2_hbm, src_hbm, dst_hbm, gsum_hbm,
                    idxa1, idxa2, rows1, rows2, gsem0, gsem1, osem0, osem1):
    c = lax.axis_index("c")
    s = lax.axis_index("s")
    wid = c * NS + s
    epw = E // NW
    cpb = epw // CH
    nchunks = B * cpb
    gsems = (gsem0, gsem1)
    osems = (osem0, osem1)

    for b in range(B):
        pltpu.sync_copy(src_hbm.at[pl.ds(b * E + wid * epw, epw)],
                        idxa1.at[pl.ds(b * epw, epw)])
        pltpu.sync_copy(dst_hbm.at[pl.ds(b * E + wid * epw, epw)],
                        idxa2.at[pl.ds(b * epw, epw)])
    vpb = epw // L

    @plsc.parallel_loop(vpb, B * vpb, unroll=4)
    def _offset(j):
        boff = (j // vpb) * N
        sl = pl.ds(j * L, L)
        idxa1[sl] = idxa1[sl] + boff
        idxa2[sl] = idxa2[sl] + boff

    def prep(i, k, wait_out):
        if wait_out:
            pltpu.make_async_copy(rows1.at[k], gsum_hbm.at[pl.ds(0, CH)],
                                  osems[k]).wait()
        pltpu.async_copy(p1_hbm.at[idxa1.at[pl.ds(i * CH, CH)]],
                         rows1.at[k], gsems[k])
        pltpu.async_copy(p2_hbm.at[idxa2.at[pl.ds(i * CH, CH)]],
                         rows2.at[k], gsems[k])

    def consume(i, k):
        pltpu.make_async_copy(p1_hbm.at[idxa1.at[pl.ds(0, CH)]],
                              rows1.at[k], gsems[k]).wait()
        pltpu.make_async_copy(p2_hbm.at[idxa2.at[pl.ds(0, CH)]],
                              rows2.at[k], gsems[k]).wait()

        @plsc.parallel_loop(0, CH, unroll=2)
        def _addrow(r):
            for j in range(D // L):
                sl = pl.ds(j * L, L)
                rows1[k, r, sl] = rows1[k, r, sl] + rows2[k, r, sl]

        b = i // cpb
        base = b * E + wid * epw + (i - b * cpb) * CH
        pltpu.async_copy(rows1.at[k], gsum_hbm.at[pl.ds(base, CH)], osems[k])

    prep(0, 0, False)
    prep(1, 1, False)

    def step(si, carry):
        c0 = 2 * si
        consume(c0, 0)

        @pl.when(si < nchunks // 2 - 1)
        def _():
            prep(c0 + 2, 0, True)

        consume(c0 + 1, 1)

        @pl.when(si < nchunks // 2 - 1)
        def _():
            prep(c0 + 3, 1, True)

        return carry

    lax.fori_loop(0, nchunks // 2, step, 0)
    pltpu.make_async_copy(rows1.at[0], gsum_hbm.at[pl.ds(0, CH)], osem0).wait()
    pltpu.make_async_copy(rows1.at[1], gsum_hbm.at[pl.ds(0, CH)], osem1).wait()


def _sc_gather(p1, p2, src, dst):
    mesh = plsc.VectorSubcoreMesh(core_axis_name="c", subcore_axis_name="s")
    return pl.kernel(
        _sc_gather_body,
        out_type=jax.ShapeDtypeStruct((B * E, D), _F32),
        mesh=mesh,
        scratch_types=[
            pltpu.VMEM((B * (E // NW),), jnp.int32),
            pltpu.VMEM((B * (E // NW),), jnp.int32),
            pltpu.VMEM((2, CH, D), _F32),
            pltpu.VMEM((2, CH, D), _F32),
            pltpu.SemaphoreType.DMA,
            pltpu.SemaphoreType.DMA,
            pltpu.SemaphoreType.DMA,
            pltpu.SemaphoreType.DMA,
        ],
        compiler_params=pltpu.CompilerParams(needs_layout_passes=False),
        name="sc_gather_sum",
    )(p1, p2, src, dst)


_BE = 4096


def _edge_mlp_body(g_ref, ef_ref, w_ref, bm_ref, g1_ref, b1_ref, wc_ref, out_ref):
    w3 = w_ref[2 * D:2 * D + DE, :]
    x = g_ref[...] + jnp.dot(ef_ref[...], w3, preferred_element_type=_F32)
    x = x + bm_ref[...]
    x = _gelu(x)
    x = _layernorm(x, g1_ref[...], b1_ref[...])
    out_ref[...] = x * wc_ref[...]


def _edge_mlp(gsum, eff, W_msg, b_msg, ln1_g, ln1_b, wcol):
    grid = (B * E // _BE,)
    return pl.pallas_call(
        _edge_mlp_body,
        grid=grid,
        in_specs=[
            pl.BlockSpec((_BE, D), lambda i: (i, 0)),
            pl.BlockSpec((_BE, DE), lambda i: (i, 0)),
            pl.BlockSpec((2 * D + DE, D), lambda i: (0, 0)),
            pl.BlockSpec((D,), lambda i: (0,)),
            pl.BlockSpec((D,), lambda i: (0,)),
            pl.BlockSpec((D,), lambda i: (0,)),
            pl.BlockSpec((_BE, 1), lambda i: (i, 0)),
        ],
        out_specs=pl.BlockSpec((_BE, D), lambda i: (i, 0)),
        out_shape=jax.ShapeDtypeStruct((B * E, D), _F32),
        name="edge_mlp",
    )(gsum, eff, W_msg, b_msg, ln1_g, ln1_b, wcol)


def _sc_scatter_body(w_hbm, seg_hbm, dst2_hbm, agg_hbm,
                     acc, dstt, segb, comb, rows, zb,
                     rsem0, rsem1, ssem0, ssem1):
    c = lax.axis_index("c")
    sid = lax.axis_index("s")
    rsems = (rsem0, rsem1)
    ssems = (ssem0, ssem1)

    def zrow(r, carry):
        for k in range(D // L):
            zb[r, pl.ds(k * L, L)] = jnp.zeros((L,), _F32)
        return carry

    lax.fori_loop(0, 64, zrow, 0)
    pltpu.sync_copy(zb, acc.at[pl.ds(sid * 128, 64)])
    pltpu.sync_copy(zb, acc.at[pl.ds(sid * 128 + 64, 64)])
    plsc.subcore_barrier()

    eps = E // NS
    cpb = eps // CH
    for t in range(2):
        b = 2 * c + t
        pltpu.sync_copy(dst2_hbm.at[pl.ds(b * S, S)], dstt)

        def base_of(ci):
            return b * E + sid * eps + ci * CH

        def prep(ci, k, wait_scatter):
            if wait_scatter:
                pltpu.make_async_copy(rows.at[k], acc.at[comb.at[k]],
                                      ssems[k]).wait()
            base = base_of(ci)
            pltpu.sync_copy(seg_hbm.at[pl.ds(base, CH)], segb.at[k])
            pltpu.async_copy(w_hbm.at[pl.ds(base, CH)], rows.at[k], rsems[k])

        def consume(k):
            for j in range(CH // L):
                sl = pl.ds(j * L, L)
                sv = segb[k, sl]
                cv = plsc.load_gather(dstt, [sv])
                comb[k, sl] = cv + (t * N)
            pltpu.make_async_copy(w_hbm.at[pl.ds(0, CH)], rows.at[k],
                                  rsems[k]).wait()
            pltpu.async_copy(rows.at[k], acc.at[comb.at[k]], ssems[k], add=True)

        prep(0, 0, False)
        prep(1, 1, False)

        def step(si, carry):
            consume(0)

            @pl.when(si < cpb // 2 - 1)
            def _():
                prep(2 * si + 2, 0, True)

            consume(1)

            @pl.when(si < cpb // 2 - 1)
            def _():
                prep(2 * si + 3, 1, True)

            return carry

        lax.fori_loop(0, cpb // 2, step, 0)
        pltpu.make_async_copy(rows.at[0], acc.at[comb.at[0]], ssem0).wait()
        pltpu.make_async_copy(rows.at[1], acc.at[comb.at[1]], ssem1).wait()

    plsc.subcore_barrier()
    pltpu.sync_copy(acc.at[pl.ds(sid * 128, 128)], rows.at[0])
    pltpu.sync_copy(rows.at[0], agg_hbm.at[pl.ds(2 * c * N + sid * 128, 128)])


def _sc_scatter(wflat, seg, dst2):
    mesh = plsc.VectorSubcoreMesh(core_axis_name="c", subcore_axis_name="s")
    return pl.kernel(
        _sc_scatter_body,
        out_type=jax.ShapeDtypeStruct((B * N, D), _F32),
        mesh=mesh,
        scratch_types=[
            pltpu.VMEM_SHARED((2 * N, D), _F32),
            pltpu.VMEM((S,), jnp.int32),
            pltpu.VMEM((2, CH), jnp.int32),
            pltpu.VMEM((2, CH), jnp.int32),
            pltpu.VMEM((2, CH, D), _F32),
            pltpu.VMEM((64, D), _F32),
            pltpu.SemaphoreType.DMA,
            pltpu.SemaphoreType.DMA,
            pltpu.SemaphoreType.DMA,
            pltpu.SemaphoreType.DMA,
        ],
        compiler_params=pltpu.CompilerParams(needs_layout_passes=False),
        name="sc_scatter_agg",
    )(wflat, seg, dst2)


def _attn_body(n_ref, a_ref, wq_ref, bq_ref, wk_ref, bk_ref, wv_ref, bv_ref,
               wc_ref, bc_ref, g2_ref, b2_ref, out_ref):
    xc = jnp.concatenate([n_ref[0], a_ref[0]], axis=1)
    q = jnp.dot(xc, wq_ref[...], preferred_element_type=_F32) + bq_ref[...]
    k = jnp.dot(xc, wk_ref[...], preferred_element_type=_F32) + bk_ref[...]
    v = jnp.dot(xc, wv_ref[...], preferred_element_type=_F32) + bv_ref[...]
    li = lax.broadcasted_iota(jnp.int32, (N, D), 1)
    att = jnp.zeros((N, D), _F32)
    inv_sqrt = 1.0 / (PH ** 0.5)
    for h in range(H):
        mh = (li // PH) == h
        qm = jnp.where(mh, q, 0.0)
        score = lax.dot_general(qm, k, (((1,), (1,)), ((), ())),
                                preferred_element_type=_F32) * inv_sqrt
        mx = jnp.max(score, axis=-1, keepdims=True)
        p = jnp.exp(score - mx)
        attw = p / jnp.sum(p, axis=-1, keepdims=True)
        vm = jnp.where(mh, v, 0.0)
        att = att + lax.dot_general(attw, vm, (((1,), (0,)), ((), ())),
                                    preferred_element_type=_F32)
    u = jnp.dot(att, wc_ref[...], preferred_element_type=_F32) + bc_ref[...]
    u = _gelu(u)
    out_ref[0] = _layernorm(u, g2_ref[...], b2_ref[...])


def _attention(nodes, agg, Wq, bq, Wk, bk, Wv, bv, Wc, bc, ln2_g, ln2_b):
    vec = lambda: pl.BlockSpec((D,), lambda i: (0,))
    mat2 = lambda: pl.BlockSpec((2 * D, D), lambda i: (0, 0))
    return pl.pallas_call(
        _attn_body,
        grid=(B,),
        in_specs=[
            pl.BlockSpec((1, N, D), lambda i: (i, 0, 0)),
            pl.BlockSpec((1, N, D), lambda i: (i, 0, 0)),
            mat2(), vec(), mat2(), vec(), mat2(), vec(),
            pl.BlockSpec((D, D), lambda i: (0, 0)), vec(),
            vec(), vec(),
        ],
        out_specs=pl.BlockSpec((1, N, D), lambda i: (i, 0, 0)),
        out_shape=jax.ShapeDtypeStruct((B, N, D), _F32),
        name="attention_update",
    )(nodes, agg, Wq, bq, Wk, bk, Wv, bv, Wc, bc, ln2_g, ln2_b)


def kernel(nodes, edge_features, edges, edge_weights, W_msg, b_msg,
           ln1_g, ln1_b, Wq, bq, Wk, bk, Wv, bv, Wc, bc, ln2_g, ln2_b):
    e32 = edges.astype(jnp.int32)
    src = e32[:, :, 0].reshape(B * E)
    dst = e32[:, :, 1].reshape(B * E)
    dst2 = e32[:, :S, 1].reshape(B * S)
    seg = edge_weights[:, :, 0].astype(jnp.int32).reshape(B * E)
    wcol = edge_weights[:, :, 1].reshape(B * E, 1)
    eff = edge_features.reshape(B * E, DE)
    nf = nodes.reshape(B * N, D)

    p1, p2 = _proj(nf, W_msg)
    gsum = _sc_gather(p1, p2, src, dst)
    wflat = _edge_mlp(gsum, eff, W_msg, b_msg, ln1_g, ln1_b, wcol)
    agg = _sc_scatter(wflat, seg, dst2)
    updated = _attention(nodes, agg.reshape(B, N, D),
                         Wq, bq, Wk, bk, Wv, bv, Wc, bc, ln2_g, ln2_b)
    return (updated, wflat.reshape(B, E, D), edges, edge_weights)
